# Initial kernel scaffold; baseline (speedup 1.0000x reference)
import jax
import jax.numpy as jnp
from jax.experimental import pallas as pl


def kernel(x, emb):
    raise NotImplementedError("write your pallas kernel here")


# speedup vs baseline: 1.5942x; 1.5942x over previous
import jax
import jax.numpy as jnp
from jax.experimental import pallas as pl


def _bcast_kernel(emb_ref, out_ref):
    out_ref[...] = jnp.broadcast_to(emb_ref[...][None], out_ref.shape)


def kernel(x, emb):
    b, s, d = x.shape
    BS = 512
    grid = (s // BS,)
    return pl.pallas_call(
        _bcast_kernel,
        grid=grid,
        in_specs=[pl.BlockSpec((BS, d), lambda i: (i, 0))],
        out_specs=pl.BlockSpec((b, BS, d), lambda i: (0, i, 0)),
        out_shape=jax.ShapeDtypeStruct((b, s, d), x.dtype),
    )(emb[:s])
